# baseline (speedup 1.0000x reference)
import functools

import jax
import jax.numpy as jnp
from jax import lax
from jax.experimental import pallas as pl
from jax.experimental.pallas import tpu as pltpu
from jax.experimental.pallas import tpu_sc as plsc

B, F, D = 16384, 26, 32
CARD = 100000
NJ = (CARD * F + 127) // 128
ROWS_W = NJ * 32
NW = 32
BW = B // NW
SB = 128
NSB = BW // SB
KV = SB // 16


NJ_FULL = NJ - 1
CB = 4
NCHT = NJ_FULL // CB
CPW = (NCHT + NW - 1) // NW


@functools.lru_cache(maxsize=1)
def _build_widen():
    mesh = plsc.VectorSubcoreMesh(core_axis_name="c", subcore_axis_name="s")
    return functools.partial(
        pl.kernel,
        out_type=jax.ShapeDtypeStruct((ROWS_W, 128), jnp.float32),
        mesh=mesh,
        scratch_types=[
            pltpu.VMEM((D, CB * 128), jnp.float32),
            pltpu.VMEM((D, CB * 128), jnp.float32),
            pltpu.VMEM((CB * 32, 128), jnp.float32),
            pltpu.VMEM((CB * 32, 128), jnp.float32),
            pltpu.VMEM((64, 128), jnp.float32),
            pltpu.SemaphoreType.DMA,
            pltpu.SemaphoreType.DMA,
        ],
        compiler_params=pltpu.CompilerParams(needs_layout_passes=False),
    )(_widen_body)


def _widen_body(tt_hbm, tail_hbm, tw_hbm, in0_v, in1_v, out0_v, out1_v,
                tail_v, isem, osem):
    wid = lax.axis_index("s") * 2 + lax.axis_index("c")
    ins = [in0_v, in1_v]
    outs = [out0_v, out1_v]

    def chunk_id(i):
        return jnp.minimum(wid + i * NW, NCHT - 1)

    def fire_in(i, p):
        pltpu.async_copy(
            tt_hbm.at[:, pl.ds(chunk_id(i) * (CB * 128), CB * 128)],
            ins[p], isem)

    def drain_in(p):
        pltpu.make_async_copy(tt_hbm.at[:, pl.ds(0, CB * 128)],
                              ins[p], isem).wait()

    def fire_out(i, p):
        pltpu.async_copy(
            outs[p], tw_hbm.at[pl.ds(chunk_id(i) * (CB * 32), CB * 32)],
            osem)

    def drain_out(p):
        pltpu.make_async_copy(tw_hbm.at[pl.ds(0, CB * 32)],
                              outs[p], osem).wait()

    def transpose(p):
        def wg_body(wg, carry):
            for dw in range(8):
                w = wg * 8 + dw
                cols0 = jnp.full((16,), w, jnp.int32)
                for s in range(CB):
                    for h in range(8):
                        rows = lax.iota(jnp.int32, 16) + (h % 2) * 16
                        e = plsc.load_gather(
                            ins[p], [rows, cols0 + (s * 128 + 32 * (h // 2))])
                        outs[p][s * 32 + w, pl.ds(h * 16, 16)] = e
            return carry
        lax.fori_loop(0, 4, wg_body, 0)

    @pl.when(wid == 0)
    def _tail():
        pltpu.sync_copy(tail_hbm, tail_v)

        def tw_body(w, carry):
            for h in range(4):
                out0_v[w, pl.ds(h * 16, 16)] = (
                    tail_v[32 * (h // 2) + w, pl.ds((h % 2) * 16, 16)])
            return carry

        lax.fori_loop(0, 32, tw_body, 0)
        pltpu.sync_copy(out0_v.at[pl.ds(0, 32)],
                        tw_hbm.at[pl.ds(NJ_FULL * 32, 32)])

    fire_in(0, 0)
    fire_in(1, 1)

    def pair_body(i2, carry):
        for p in range(2):
            i = i2 * 2 + p
            drain_in(p)

            @pl.when(i >= 2)
            def _do():
                drain_out(p)

            transpose(p)
            fire_out(i, p)

            @pl.when(i + 2 < CPW)
            def _pre():
                fire_in(i + 2, p)
        return carry

    lax.fori_loop(0, CPW // 2, pair_body, 0)
    i_last = CPW - 1
    drain_in(0)
    drain_out(0)
    transpose(0)
    fire_out(i_last, 0)
    drain_out(1)
    drain_out(0)


def _widen(tt, tail):
    return _build_widen()(tt, tail)


@functools.lru_cache(maxsize=1)
def _build():
    mesh = plsc.VectorSubcoreMesh(core_axis_name="c", subcore_axis_name="s")
    return functools.partial(
        pl.kernel,
        out_type=jax.ShapeDtypeStruct((F, D, B), jnp.float32),
        mesh=mesh,
        scratch_types=[
            pltpu.VMEM((BW,), jnp.int32),
            pltpu.VMEM((SB,), jnp.int32),
            pltpu.VMEM((SB,), jnp.int32),
            pltpu.VMEM((SB, 128), jnp.float32),
            pltpu.VMEM((D, SB), jnp.float32),
            pltpu.VMEM((F * D,), jnp.float32),
            pltpu.VMEM((F * D * 16,), jnp.float32),
            pltpu.SemaphoreType.DMA,
        ],
        compiler_params=pltpu.CompilerParams(needs_layout_passes=False),
    )(_embed_body)


def _embed_body(xtf_hbm, tablew_hbm, shared_hbm, out_hbm,
                xf_v, widx_v, sub_v, wide_v, outb_v, sh_v, shb_v, sem):
    wid = lax.axis_index("s") * 2 + lax.axis_index("c")
    b0 = wid * BW

    pltpu.sync_copy(shared_hbm, sh_v)

    def splat_body(j, carry):
        e = plsc.load_gather(sh_v, [jnp.full((16,), j, jnp.int32)])
        shb_v[pl.ds(j * 16, 16)] = e
        return carry

    lax.fori_loop(0, F * D, splat_body, 0)

    def field_body(f, carry):
        off = f * CARD
        pltpu.sync_copy(xtf_hbm.at[pl.ds(f * B + b0, BW)], xf_v)

        for sb in range(NSB):
            for k in range(KV):
                v = xf_v[pl.ds(sb * SB + k * 16, 16)] + off
                widx_v[pl.ds(k * 16, 16)] = ((v >> 7) << 5) | (v & 31)
                sub_v[pl.ds(k * 16, 16)] = v & 96

            pltpu.async_copy(tablew_hbm.at[widx_v], wide_v, sem).wait()

            def extract_body(k, carry2):
                rows = lax.iota(jnp.int32, 16) + k * 16
                cols = sub_v[pl.ds(k * 16, 16)]
                for d in range(D):
                    e = plsc.load_gather(wide_v, [rows, cols + d])
                    o = e + shb_v[pl.ds((f * D + d) * 16, 16)]
                    outb_v[d, pl.ds(k * 16, 16)] = o
                return carry2

            lax.fori_loop(0, KV, extract_body, 0)

            pltpu.sync_copy(outb_v, out_hbm.at[f, :, pl.ds(b0 + sb * SB, SB)])
        return carry

    lax.fori_loop(0, F, field_body, 0)


def kernel(x, indiv_embed, shared_embed):
    xtf = x.T.reshape(F * B)
    tail = jnp.pad(indiv_embed[NJ_FULL * 128:], ((0, 0), (0, 128 - D)))
    tw = _widen(indiv_embed.T, tail)
    outp = _build()(xtf, tw, shared_embed.reshape(F * D))
    return jnp.transpose(outp, (2, 0, 1))


# speedup vs baseline: 1.5052x; 1.5052x over previous
import functools

import jax
import jax.numpy as jnp
from jax import lax
from jax.experimental import pallas as pl
from jax.experimental.pallas import tpu as pltpu
from jax.experimental.pallas import tpu_sc as plsc

B, F, D = 16384, 26, 32
CARD = 100000
TOTAL = B * F
NW = 32
RW = TOTAL // NW
CHUNK = 1664
NCHUNK = RW // CHUNK
GPC = CHUNK // 128
PER = 208

@functools.lru_cache(maxsize=1)
def _build():
    mesh = plsc.VectorSubcoreMesh(core_axis_name="c", subcore_axis_name="s")
    return functools.partial(
        pl.kernel,
        out_type=jax.ShapeDtypeStruct((TOTAL, D), jnp.float32),
        mesh=mesh,
        scratch_types=[
            pltpu.VMEM((CHUNK,), jnp.int32),
            pltpu.VMEM((CHUNK, D), jnp.float32),
            pltpu.VMEM((PER,), jnp.int32),
            pltpu.VMEM((F * D,), jnp.float32),
            pltpu.SemaphoreType.DMA,
        ],
        compiler_params=pltpu.CompilerParams(use_tc_tiling_on_sc=False),
    )(_embed_body)


def _embed_body(x_hbm, table_hbm, shared_hbm, out_hbm,
                idx_v, rows_v, offs_v, shared_v, sem):
    wid = lax.axis_index("s") * 2 + lax.axis_index("c")
    base = wid * RW

    pltpu.sync_copy(shared_hbm, shared_v)

    for k in range(PER // 16):
        v = lax.iota(jnp.int32, 16) + (16 * k)
        offs_v[pl.ds(16 * k, 16)] = (v % 26) * CARD

    def chunk_body(c, carry):
        start = base + c * CHUNK

        pltpu.sync_copy(x_hbm.at[pl.ds(start, CHUNK)], idx_v)

        for v in range(CHUNK // 16):
            idx_v[pl.ds(16 * v, 16)] = (
                idx_v[pl.ds(16 * v, 16)]
                + offs_v[pl.ds((16 * v) % PER, 16)])

        descs = [
            pltpu.async_copy(table_hbm.at[idx_v.at[pl.ds(j * 128, 128)]],
                             rows_v.at[pl.ds(j * 128, 128)], sem)
            for j in range(GPC)
        ]
        for d in descs:
            d.wait()

        def group_body(g, carry2):
            row0 = g * 26
            for r in range(26):
                for h in range(2):
                    rows_v[row0 + r, pl.ds(h * 16, 16)] = (
                        rows_v[row0 + r, pl.ds(h * 16, 16)]
                        + shared_v[pl.ds(r * D + h * 16, 16)])
            return carry2

        lax.fori_loop(0, CHUNK // 26, group_body, 0)

        pltpu.sync_copy(rows_v, out_hbm.at[pl.ds(start, CHUNK)])
        return carry

    lax.fori_loop(0, NCHUNK, chunk_body, 0)


def kernel(x, indiv_embed, shared_embed):
    x1 = x.reshape(TOTAL)
    sh = shared_embed.reshape(F * D)
    out = _build()(x1, indiv_embed, sh)
    return out.reshape(B, F, D)
